# Initial kernel scaffold; baseline (speedup 1.0000x reference)
import jax
import jax.numpy as jnp
from jax.experimental import pallas as pl


def kernel(x, edge_index, edge_attr, u, batch, W1, b1, g1, beta1, W2, b2, g2, beta2, W3, b3):
    raise NotImplementedError("write your pallas kernel here")


# speedup vs baseline: 10.0761x; 10.0761x over previous
import functools

import jax
import jax.numpy as jnp
from jax import lax
from jax.experimental import pallas as pl
from jax.experimental.pallas import tpu as pltpu
from jax.experimental.pallas import tpu_sc as plsc

N_NODES = 10000
N_EDGES = 320000
N_GRAPHS = 512
D_FEAT = 128
D_EDGE = 16
N_TILES = 32

CH = 128
E_CHUNKS = N_EDGES // CH
E_BASE = E_CHUNKS // N_TILES
E_EXTRA = E_CHUNKS - E_BASE * N_TILES
N_FULL = N_NODES // CH
N_REM = N_NODES - N_FULL * CH


def _sc_body(x_hbm, batch_hbm, col_hbm, ea_hbm, node_out, edge_out,
             accn_sp, acce_sp, batch_v, col_v, seg_v, rows_v,
             bidx_v, xv, bidx16_v, x16_v, zn_v, ze_v):
    cid = lax.axis_index("c")
    sid = lax.axis_index("s")
    wid = cid * 16 + sid

    zero16 = jnp.zeros((16,), jnp.float32)
    for r in range(32):
        for c in range(D_FEAT // 16):
            zn_v[r, pl.ds(c * 16, 16)] = zero16
        ze_v[r, pl.ds(0, 16)] = zero16
    pltpu.sync_copy(zn_v, accn_sp.at[pl.ds(sid * 32, 32)])
    pltpu.sync_copy(ze_v, acce_sp.at[pl.ds(sid * 32, 32)])

    pltpu.sync_copy(batch_hbm, batch_v)
    plsc.subcore_barrier()

    for k in range(3):
        n = wid + N_TILES * k

        @pl.when(n < N_FULL)
        def _():
            base = n * CH
            pltpu.sync_copy(batch_hbm.at[pl.ds(base, CH)], bidx_v)
            pltpu.sync_copy(x_hbm.at[pl.ds(base, CH), :], xv)
            pltpu.sync_copy(xv, accn_sp.at[bidx_v], add=True)

    @pl.when(wid == 14)
    def _():
        base = N_FULL * CH
        pltpu.sync_copy(batch_hbm.at[pl.ds(base, N_REM)], bidx16_v)
        pltpu.sync_copy(x_hbm.at[pl.ds(base, N_REM), :], x16_v)
        pltpu.sync_copy(x16_v, accn_sp.at[bidx16_v], add=True)

    c0 = E_BASE * wid + jnp.minimum(wid, E_EXTRA)
    nchunks = E_BASE + jnp.where(wid < E_EXTRA, 1, 0)

    def edge_body(i, carry):
        base = (c0 + i) * CH
        pltpu.sync_copy(col_hbm.at[pl.ds(base, CH)], col_v)
        pltpu.sync_copy(ea_hbm.at[pl.ds(base, CH), :], rows_v)
        for g in range(CH // 16):
            idx = col_v[pl.ds(g * 16, 16)]
            seg_v[pl.ds(g * 16, 16)] = plsc.load_gather(batch_v, [idx])
        pltpu.sync_copy(rows_v, acce_sp.at[seg_v], add=True)
        return carry

    lax.fori_loop(0, nchunks, edge_body, 0)

    plsc.subcore_barrier()
    pltpu.sync_copy(accn_sp.at[pl.ds(sid * 32, 32)],
                    node_out.at[cid, pl.ds(sid * 32, 32)])
    pltpu.sync_copy(acce_sp.at[pl.ds(sid * 32, 32)],
                    edge_out.at[cid, pl.ds(sid * 32, 32)])


_sc_aggregate = pl.kernel(
    _sc_body,
    out_type=(
        jax.ShapeDtypeStruct((2, N_GRAPHS, D_FEAT), jnp.float32),
        jax.ShapeDtypeStruct((2, N_GRAPHS, D_EDGE), jnp.float32),
    ),
    mesh=plsc.VectorSubcoreMesh(core_axis_name="c", subcore_axis_name="s"),
    compiler_params=pltpu.CompilerParams(needs_layout_passes=False,
                                         use_tc_tiling_on_sc=False),
    scratch_types=[
        pltpu.VMEM_SHARED((N_GRAPHS, D_FEAT), jnp.float32),
        pltpu.VMEM_SHARED((N_GRAPHS, D_EDGE), jnp.float32),
        pltpu.VMEM((N_NODES,), jnp.int32),
        pltpu.VMEM((CH,), jnp.int32),
        pltpu.VMEM((CH,), jnp.int32),
        pltpu.VMEM((CH, D_EDGE), jnp.float32),
        pltpu.VMEM((CH,), jnp.int32),
        pltpu.VMEM((CH, D_FEAT), jnp.float32),
        pltpu.VMEM((N_REM,), jnp.int32),
        pltpu.VMEM((N_REM, D_FEAT), jnp.float32),
        pltpu.VMEM((32, D_FEAT), jnp.float32),
        pltpu.VMEM((32, D_EDGE), jnp.float32),
    ],
)


def _mlp_body(np_ref, ep_ref, w1_ref, b1_ref, g1_ref, bt1_ref,
              w2_ref, b2_ref, g2_ref, bt2_ref, w3_ref, b3_ref, o_ref):
    f32 = jnp.float32
    hi = jax.lax.Precision.HIGHEST
    node = np_ref[0] + np_ref[1]
    edge = ep_ref[0] + ep_ref[1]
    w1 = w1_ref[...]
    h = (jnp.dot(node, w1[:D_FEAT], preferred_element_type=f32, precision=hi)
         + jnp.dot(edge, w1[D_FEAT:], preferred_element_type=f32, precision=hi)
         + b1_ref[...])
    h = jnp.where(h >= 0, h, 0.01 * h)
    mean = jnp.mean(h, axis=0, keepdims=True)
    d = h - mean
    var = jnp.mean(d * d, axis=0, keepdims=True)
    h = g1_ref[...] * d / jnp.sqrt(var + 1e-5) + bt1_ref[...]

    h = jnp.dot(h, w2_ref[...], preferred_element_type=f32, precision=hi) + b2_ref[...]
    h = jnp.where(h >= 0, h, 0.01 * h)
    mean = jnp.mean(h, axis=0, keepdims=True)
    d = h - mean
    var = jnp.mean(d * d, axis=0, keepdims=True)
    h = g2_ref[...] * d / jnp.sqrt(var + 1e-5) + bt2_ref[...]

    o_ref[...] = jnp.dot(h, w3_ref[...], preferred_element_type=f32,
                         precision=hi) + b3_ref[...]


_mlp_head = pl.pallas_call(
    _mlp_body,
    out_shape=jax.ShapeDtypeStruct((N_GRAPHS, D_FEAT), jnp.float32),
)


@functools.partial(jax.jit, static_argnames=())
def kernel(x, edge_index, edge_attr, u, batch, W1, b1, g1, beta1,
           W2, b2, g2, beta2, W3, b3):
    col = edge_index[1].astype(jnp.int32)
    batch32 = batch.astype(jnp.int32)
    node_part, edge_part = _sc_aggregate(x, batch32, col, edge_attr)
    return _mlp_head(node_part, edge_part,
                     W1, b1.reshape(1, -1), g1.reshape(1, -1),
                     beta1.reshape(1, -1),
                     W2, b2.reshape(1, -1), g2.reshape(1, -1),
                     beta2.reshape(1, -1),
                     W3, b3.reshape(1, -1))


# speedup vs baseline: 20.6746x; 2.0518x over previous
import functools

import jax
import jax.numpy as jnp
from jax import lax
from jax.experimental import pallas as pl
from jax.experimental.pallas import tpu as pltpu
from jax.experimental.pallas import tpu_sc as plsc

N_NODES = 10000
N_EDGES = 320000
N_GRAPHS = 512
D_FEAT = 128
D_EDGE = 16
N_TILES = 32

NCH = 128
N_FULL = N_NODES // NCH
N_REM = N_NODES - N_FULL * NCH

EK = 2000
E_PER_TILE = N_EDGES // N_TILES
E_CHUNKS = E_PER_TILE // EK
EG = EK // 16


def _sc_body(x_hbm, batch_hbm, col_hbm, eat_hbm, node_out, edge_out,
             accn_sp, acce_sp, batch_v, col_v, et_v, acc2_v,
             bidx_v, xv, bidx16_v, x16_v, zn_v, ze_v, ident_v):
    cid = lax.axis_index("c")
    sid = lax.axis_index("s")
    wid = cid * 16 + sid

    zero16 = jnp.zeros((16,), jnp.float32)
    for r in range(D_EDGE):
        for c in range(N_GRAPHS // 16):
            acc2_v[r, pl.ds(c * 16, 16)] = zero16
    for r in range(32):
        for c in range(D_FEAT // 16):
            zn_v[r, pl.ds(c * 16, 16)] = zero16
    for c in range(N_GRAPHS // 16):
        ze_v[pl.ds(c * 16, 16)] = zero16
    ident_v[pl.ds(0, 16)] = jnp.arange(16, dtype=jnp.int32)
    pltpu.sync_copy(zn_v, accn_sp.at[pl.ds(sid * 32, 32)])
    pltpu.sync_copy(ze_v, acce_sp.at[sid])

    pltpu.sync_copy(batch_hbm, batch_v)
    plsc.subcore_barrier()

    for k in range(3):
        n = wid + N_TILES * k

        @pl.when(n < N_FULL)
        def _():
            base = n * NCH
            pltpu.sync_copy(batch_hbm.at[pl.ds(base, NCH)], bidx_v)
            pltpu.sync_copy(x_hbm.at[pl.ds(base, NCH), :], xv)
            pltpu.sync_copy(xv, accn_sp.at[bidx_v], add=True)

    @pl.when(wid == 14)
    def _():
        base = N_FULL * NCH
        pltpu.sync_copy(batch_hbm.at[pl.ds(base, N_REM)], bidx16_v)
        pltpu.sync_copy(x_hbm.at[pl.ds(base, N_REM), :], x16_v)
        pltpu.sync_copy(x16_v, accn_sp.at[bidx16_v], add=True)

    def edge_chunk(c, carry):
        base = wid * E_PER_TILE + c * EK
        pltpu.sync_copy(col_hbm.at[pl.ds(base, EK)], col_v)
        pltpu.sync_copy(eat_hbm.at[:, pl.ds(base, EK)], et_v)

        def group(j, carry2):
            idx = col_v[pl.ds(j * 16, 16)]
            sv = plsc.load_gather(batch_v, [idx])
            for f in range(D_EDGE):
                vals = et_v[f, pl.ds(j * 16, 16)]
                fvec = jnp.full((16,), f, jnp.int32)
                plsc.addupdate_scatter(acc2_v, [fvec, sv], vals)
            return carry2

        lax.fori_loop(0, EG, group, 0)
        return carry

    lax.fori_loop(0, E_CHUNKS, edge_chunk, 0)

    pltpu.sync_copy(acc2_v, acce_sp.at[ident_v], add=True)

    plsc.subcore_barrier()
    pltpu.sync_copy(accn_sp.at[pl.ds(sid * 32, 32)],
                    node_out.at[cid, pl.ds(sid * 32, 32)])
    pltpu.sync_copy(acce_sp.at[sid], edge_out.at[cid, sid])


_sc_aggregate = pl.kernel(
    _sc_body,
    out_type=(
        jax.ShapeDtypeStruct((2, N_GRAPHS, D_FEAT), jnp.float32),
        jax.ShapeDtypeStruct((2, D_EDGE, N_GRAPHS), jnp.float32),
    ),
    mesh=plsc.VectorSubcoreMesh(core_axis_name="c", subcore_axis_name="s"),
    compiler_params=pltpu.CompilerParams(needs_layout_passes=False,
                                         use_tc_tiling_on_sc=False),
    scratch_types=[
        pltpu.VMEM_SHARED((N_GRAPHS, D_FEAT), jnp.float32),
        pltpu.VMEM_SHARED((D_EDGE, N_GRAPHS), jnp.float32),
        pltpu.VMEM((N_NODES,), jnp.int32),
        pltpu.VMEM((EK,), jnp.int32),
        pltpu.VMEM((D_EDGE, EK), jnp.float32),
        pltpu.VMEM((D_EDGE, N_GRAPHS), jnp.float32),
        pltpu.VMEM((NCH,), jnp.int32),
        pltpu.VMEM((NCH, D_FEAT), jnp.float32),
        pltpu.VMEM((N_REM,), jnp.int32),
        pltpu.VMEM((N_REM, D_FEAT), jnp.float32),
        pltpu.VMEM((32, D_FEAT), jnp.float32),
        pltpu.VMEM((N_GRAPHS,), jnp.float32),
        pltpu.VMEM((16,), jnp.int32),
    ],
)


def _mlp_body(np_ref, ep_ref, w1_ref, b1_ref, g1_ref, bt1_ref,
              w2_ref, b2_ref, g2_ref, bt2_ref, w3_ref, b3_ref, o_ref):
    f32 = jnp.float32
    hi = jax.lax.Precision.HIGHEST
    node = np_ref[0] + np_ref[1]
    edge_t = ep_ref[0] + ep_ref[1]
    w1 = w1_ref[...]
    h = (jnp.dot(node, w1[:D_FEAT], preferred_element_type=f32, precision=hi)
         + lax.dot_general(edge_t, w1[D_FEAT:],
                           (((0,), (0,)), ((), ())),
                           preferred_element_type=f32, precision=hi)
         + b1_ref[...])
    h = jnp.where(h >= 0, h, 0.01 * h)
    mean = jnp.mean(h, axis=0, keepdims=True)
    d = h - mean
    var = jnp.mean(d * d, axis=0, keepdims=True)
    h = g1_ref[...] * d / jnp.sqrt(var + 1e-5) + bt1_ref[...]

    h = jnp.dot(h, w2_ref[...], preferred_element_type=f32, precision=hi) + b2_ref[...]
    h = jnp.where(h >= 0, h, 0.01 * h)
    mean = jnp.mean(h, axis=0, keepdims=True)
    d = h - mean
    var = jnp.mean(d * d, axis=0, keepdims=True)
    h = g2_ref[...] * d / jnp.sqrt(var + 1e-5) + bt2_ref[...]

    o_ref[...] = jnp.dot(h, w3_ref[...], preferred_element_type=f32,
                         precision=hi) + b3_ref[...]


_mlp_head = pl.pallas_call(
    _mlp_body,
    out_shape=jax.ShapeDtypeStruct((N_GRAPHS, D_FEAT), jnp.float32),
)


@functools.partial(jax.jit, static_argnames=())
def kernel(x, edge_index, edge_attr, u, batch, W1, b1, g1, beta1,
           W2, b2, g2, beta2, W3, b3):
    col = edge_index[1].astype(jnp.int32)
    batch32 = batch.astype(jnp.int32)
    ea_t = edge_attr.T
    node_part, edge_part = _sc_aggregate(x, batch32, col, ea_t)
    return _mlp_head(node_part, edge_part,
                     W1, b1.reshape(1, -1), g1.reshape(1, -1),
                     beta1.reshape(1, -1),
                     W2, b2.reshape(1, -1), g2.reshape(1, -1),
                     beta2.reshape(1, -1),
                     W3, b3.reshape(1, -1))


# speedup vs baseline: 25.3060x; 1.2240x over previous
import functools

import jax
import jax.numpy as jnp
from jax import lax
from jax.experimental import pallas as pl
from jax.experimental.pallas import tpu as pltpu
from jax.experimental.pallas import tpu_sc as plsc

N_NODES = 10000
N_EDGES = 320000
N_GRAPHS = 512
D_FEAT = 128
D_EDGE = 16
N_TILES = 32

NCH = 128
N_FULL = N_NODES // NCH
N_REM = N_NODES - N_FULL * NCH

EK = 512
E_CHUNKS = N_EDGES // EK
K_MAX = -(-E_CHUNKS // N_TILES)
EG = EK // 16
EROWS = D_EDGE * N_GRAPHS // 128


def _sc_body(x_hbm, batch_hbm, col_hbm, eat_hbm, node_out, edge_out,
             accn_sp, acce_sp, batch_v, col_v, et_v, col2_v, et2_v, acc2_v,
             bidx_v, xv, bidx16_v, x16_v, zn_v, ident_v, sem0, sem1):
    cid = lax.axis_index("c")
    sid = lax.axis_index("s")
    wid = cid * 16 + sid

    zero16 = jnp.zeros((16,), jnp.float32)
    for r in range(EROWS):
        for c in range(D_FEAT // 16):
            acc2_v[r, pl.ds(c * 16, 16)] = zero16
    for r in range(32):
        for c in range(D_FEAT // 16):
            zn_v[r, pl.ds(c * 16, 16)] = zero16
    for c in range(4):
        ident_v[pl.ds(c * 16, 16)] = jnp.arange(16, dtype=jnp.int32) + c * 16
    pltpu.sync_copy(zn_v, accn_sp.at[pl.ds(sid * 32, 32)])

    @pl.when(sid < 2)
    def _():
        pltpu.sync_copy(zn_v, acce_sp.at[pl.ds(sid * 32, 32)])

    pltpu.sync_copy(batch_hbm, batch_v)
    plsc.subcore_barrier()

    for k in range(3):
        n = wid + N_TILES * k

        @pl.when(n < N_FULL)
        def _():
            base = n * NCH
            pltpu.sync_copy(batch_hbm.at[pl.ds(base, NCH)], bidx_v)
            pltpu.sync_copy(x_hbm.at[pl.ds(base, NCH), :], xv)
            pltpu.sync_copy(xv, accn_sp.at[bidx_v], add=True)

    @pl.when(wid == 14)
    def _():
        base = N_FULL * NCH
        pltpu.sync_copy(batch_hbm.at[pl.ds(base, N_REM)], bidx16_v)
        pltpu.sync_copy(x_hbm.at[pl.ds(base, N_REM), :], x16_v)
        pltpu.sync_copy(x16_v, accn_sp.at[bidx16_v], add=True)

    col_bufs = (col_v, col2_v)
    et_bufs = (et_v, et2_v)
    sems = (sem0, sem1)

    def start_round(k):
        base = (wid + N_TILES * k) * EK
        s = sems[k % 2]
        return (pltpu.async_copy(col_hbm.at[pl.ds(base, EK)],
                                 col_bufs[k % 2], s),
                pltpu.async_copy(eat_hbm.at[:, pl.ds(base, EK)],
                                 et_bufs[k % 2], s))

    def compute_round(k):
        cv, ev = col_bufs[k % 2], et_bufs[k % 2]

        def group(j, carry):
            idx = cv[pl.ds(j * 16, 16)]
            sv = plsc.load_gather(batch_v, [idx])
            srow = jax.lax.shift_right_logical(sv, 7)
            scol = jnp.bitwise_and(sv, 127)
            for f in range(D_EDGE):
                vals = ev[f, pl.ds(j * 16, 16)]
                plsc.addupdate_scatter(acc2_v, [srow + (4 * f), scol], vals)
            return carry

        lax.fori_loop(0, EG, group, 0)

    N_FULL_ROUNDS = E_CHUNKS // N_TILES
    pend = start_round(0)
    for k in range(N_FULL_ROUNDS):
        nxt = start_round(k + 1) if k + 1 < N_FULL_ROUNDS else None
        for d in pend:
            d.wait()
        compute_round(k)
        pend = nxt

    @pl.when(wid + N_TILES * N_FULL_ROUNDS < E_CHUNKS)
    def _():
        base = (wid + N_TILES * N_FULL_ROUNDS) * EK
        kb = N_FULL_ROUNDS % 2
        pltpu.sync_copy(col_hbm.at[pl.ds(base, EK)], col_bufs[kb])
        pltpu.sync_copy(eat_hbm.at[:, pl.ds(base, EK)], et_bufs[kb])
        compute_round(N_FULL_ROUNDS)

    pltpu.sync_copy(acc2_v, acce_sp.at[ident_v], add=True)

    plsc.subcore_barrier()
    pltpu.sync_copy(accn_sp.at[pl.ds(sid * 32, 32)],
                    node_out.at[cid, pl.ds(sid * 32, 32)])

    @pl.when(sid < 8)
    def _():
        pltpu.sync_copy(acce_sp.at[pl.ds(sid * 8, 8)],
                        edge_out.at[cid, pl.ds(sid * 8, 8)])


_sc_aggregate = pl.kernel(
    _sc_body,
    out_type=(
        jax.ShapeDtypeStruct((2, N_GRAPHS, D_FEAT), jnp.float32),
        jax.ShapeDtypeStruct((2, EROWS, 128), jnp.float32),
    ),
    mesh=plsc.VectorSubcoreMesh(core_axis_name="c", subcore_axis_name="s"),
    compiler_params=pltpu.CompilerParams(needs_layout_passes=False,
                                         use_tc_tiling_on_sc=True),
    scratch_types=[
        pltpu.VMEM_SHARED((N_GRAPHS, D_FEAT), jnp.float32),
        pltpu.VMEM_SHARED((EROWS, 128), jnp.float32),
        pltpu.VMEM((N_NODES,), jnp.int32),
        pltpu.VMEM((EK,), jnp.int32),
        pltpu.VMEM((D_EDGE, EK), jnp.float32),
        pltpu.VMEM((EK,), jnp.int32),
        pltpu.VMEM((D_EDGE, EK), jnp.float32),
        pltpu.VMEM((EROWS, 128), jnp.float32),
        pltpu.VMEM((NCH,), jnp.int32),
        pltpu.VMEM((NCH, D_FEAT), jnp.float32),
        pltpu.VMEM((N_REM,), jnp.int32),
        pltpu.VMEM((N_REM, D_FEAT), jnp.float32),
        pltpu.VMEM((32, D_FEAT), jnp.float32),
        pltpu.VMEM((EROWS,), jnp.int32),
        pltpu.SemaphoreType.DMA,
        pltpu.SemaphoreType.DMA,
    ],
)


def _mlp_body(np_ref, et_ref, w1_ref, b1_ref, g1_ref, bt1_ref,
              w2_ref, b2_ref, g2_ref, bt2_ref, w3_ref, b3_ref, o_ref):
    f32 = jnp.float32
    hi = jax.lax.Precision.HIGHEST
    node = np_ref[0] + np_ref[1]
    edge_t = et_ref[...]
    w1 = w1_ref[...]
    h = (jnp.dot(node, w1[:D_FEAT], preferred_element_type=f32, precision=hi)
         + lax.dot_general(edge_t, w1[D_FEAT:],
                           (((0,), (0,)), ((), ())),
                           preferred_element_type=f32, precision=hi)
         + b1_ref[...])
    h = jnp.where(h >= 0, h, 0.01 * h)
    mean = jnp.mean(h, axis=0, keepdims=True)
    d = h - mean
    var = jnp.mean(d * d, axis=0, keepdims=True)
    h = g1_ref[...] * d / jnp.sqrt(var + 1e-5) + bt1_ref[...]

    h = jnp.dot(h, w2_ref[...], preferred_element_type=f32, precision=hi) + b2_ref[...]
    h = jnp.where(h >= 0, h, 0.01 * h)
    mean = jnp.mean(h, axis=0, keepdims=True)
    d = h - mean
    var = jnp.mean(d * d, axis=0, keepdims=True)
    h = g2_ref[...] * d / jnp.sqrt(var + 1e-5) + bt2_ref[...]

    o_ref[...] = jnp.dot(h, w3_ref[...], preferred_element_type=f32,
                         precision=hi) + b3_ref[...]


_mlp_head = pl.pallas_call(
    _mlp_body,
    out_shape=jax.ShapeDtypeStruct((N_GRAPHS, D_FEAT), jnp.float32),
)


@functools.partial(jax.jit, static_argnames=())
def kernel(x, edge_index, edge_attr, u, batch, W1, b1, g1, beta1,
           W2, b2, g2, beta2, W3, b3):
    col = edge_index[1].astype(jnp.int32)
    batch32 = batch.astype(jnp.int32)
    ea_t = edge_attr.T
    node_part, edge_part = _sc_aggregate(x, batch32, col, ea_t)
    edge_t = (edge_part[0] + edge_part[1]).reshape(D_EDGE, N_GRAPHS)
    return _mlp_head(node_part, edge_t,
                     W1, b1.reshape(1, -1), g1.reshape(1, -1),
                     beta1.reshape(1, -1),
                     W2, b2.reshape(1, -1), g2.reshape(1, -1),
                     beta2.reshape(1, -1),
                     W3, b3.reshape(1, -1))


# speedup vs baseline: 26.6699x; 1.0539x over previous
import functools

import jax
import jax.numpy as jnp
from jax import lax
from jax.experimental import pallas as pl
from jax.experimental.pallas import tpu as pltpu
from jax.experimental.pallas import tpu_sc as plsc

N_NODES = 10000
N_EDGES = 320000
N_GRAPHS = 512
D_FEAT = 128
D_EDGE = 16
N_TILES = 32

NCH = 128
N_FULL = N_NODES // NCH
N_REM = N_NODES - N_FULL * NCH

EK = 512
E_CHUNKS = N_EDGES // EK
K_MAX = -(-E_CHUNKS // N_TILES)
EG = EK // 16
EROWS = D_EDGE * N_GRAPHS // 128


def _sc_body(x_hbm, batch_hbm, col_hbm, eat_hbm, node_out, edge_out,
             accn_sp, acce_sp, batch_v, col_v, et_v, col2_v, et2_v, acc2_v,
             bidx_v, xv, bidx1_v, xv1, bidx16_v, x16_v, zn_v, ident_v,
             sem0, sem1, semb, semn0, semn1, semns0, semns1):
    cid = lax.axis_index("c")
    sid = lax.axis_index("s")
    wid = cid * 16 + sid

    col_bufs = (col_v, col2_v)
    et_bufs = (et_v, et2_v)
    sems = (sem0, sem1)
    node_bufs = ((bidx_v, xv, semn0), (bidx1_v, xv1, semn1))

    def start_round(k):
        base = (wid + N_TILES * k) * EK
        s = sems[k % 2]
        return (pltpu.async_copy(col_hbm.at[pl.ds(base, EK)],
                                 col_bufs[k % 2], s),
                pltpu.async_copy(eat_hbm.at[:, pl.ds(base, EK)],
                                 et_bufs[k % 2], s))

    def start_node(k):
        bb, xb, s = node_bufs[k % 2]
        base = (wid + N_TILES * k) * NCH
        return (pltpu.async_copy(batch_hbm.at[pl.ds(base, NCH)], bb, s),
                pltpu.async_copy(x_hbm.at[pl.ds(base, NCH), :], xb, s))

    bd = pltpu.async_copy(batch_hbm, batch_v, semb)
    pend0 = start_round(0)
    pend1 = start_round(1)
    nd0 = start_node(0)

    zero16 = jnp.zeros((16,), jnp.float32)
    for r in range(EROWS):
        for c in range(D_FEAT // 16):
            acc2_v[r, pl.ds(c * 16, 16)] = zero16
    for r in range(32):
        for c in range(D_FEAT // 16):
            zn_v[r, pl.ds(c * 16, 16)] = zero16
    for c in range(4):
        ident_v[pl.ds(c * 16, 16)] = jnp.arange(16, dtype=jnp.int32) + c * 16
    pltpu.sync_copy(zn_v, accn_sp.at[pl.ds(sid * 32, 32)])

    @pl.when(sid < 2)
    def _():
        pltpu.sync_copy(zn_v, acce_sp.at[pl.ds(sid * 32, 32)])

    bd.wait()
    plsc.subcore_barrier()

    N_FULL_ROUNDS = E_CHUNKS // N_TILES
    pend = [pend0, pend1]
    nd = [nd0, None]
    ns = [None, None, None]

    def compute_round(k):
        cv, ev = col_bufs[k % 2], et_bufs[k % 2]

        def group(j, carry):
            idx = cv[pl.ds(j * 16, 16)]
            sv = plsc.load_gather(batch_v, [idx])
            srow = jax.lax.shift_right_logical(sv, 7)
            scol = jnp.bitwise_and(sv, 127)
            for f in range(D_EDGE):
                vals = ev[f, pl.ds(j * 16, 16)]
                plsc.addupdate_scatter(acc2_v, [srow + (4 * f), scol], vals)
            return carry

        lax.fori_loop(0, EG, group, 0)

    for k in range(N_FULL_ROUNDS):
        for d in pend[k % 2]:
            d.wait()
        compute_round(k)
        if k + 2 < N_FULL_ROUNDS:
            pend[k % 2] = start_round(k + 2)
        if k == 0:
            nd[1] = start_node(1)
            for d in nd[0]:
                d.wait()
            ns[0] = pltpu.async_copy(xv, accn_sp.at[bidx_v], semns0, add=True)
        elif k == 1:
            for d in nd[1]:
                d.wait()
            ns[1] = pltpu.async_copy(xv1, accn_sp.at[bidx1_v], semns1,
                                     add=True)
            ns[0].wait()

            @pl.when(wid < 14)
            def _():
                for d in start_node(2):
                    d.wait()

        elif k == 2:

            @pl.when(wid < 14)
            def _():
                pltpu.sync_copy(xv, accn_sp.at[bidx_v], add=True)

            @pl.when(wid == 14)
            def _():
                base = N_FULL * NCH
                pltpu.sync_copy(batch_hbm.at[pl.ds(base, N_REM)], bidx16_v)
                pltpu.sync_copy(x_hbm.at[pl.ds(base, N_REM), :], x16_v)
                pltpu.sync_copy(x16_v, accn_sp.at[bidx16_v], add=True)

    ns[1].wait()

    @pl.when(wid + N_TILES * N_FULL_ROUNDS < E_CHUNKS)
    def _():
        base = (wid + N_TILES * N_FULL_ROUNDS) * EK
        kb = N_FULL_ROUNDS % 2
        pltpu.sync_copy(col_hbm.at[pl.ds(base, EK)], col_bufs[kb])
        pltpu.sync_copy(eat_hbm.at[:, pl.ds(base, EK)], et_bufs[kb])
        compute_round(N_FULL_ROUNDS)

    pltpu.sync_copy(acc2_v, acce_sp.at[ident_v], add=True)

    plsc.subcore_barrier()
    pltpu.sync_copy(accn_sp.at[pl.ds(sid * 32, 32)],
                    node_out.at[cid, pl.ds(sid * 32, 32)])

    @pl.when(sid < 8)
    def _():
        pltpu.sync_copy(acce_sp.at[pl.ds(sid * 8, 8)],
                        edge_out.at[cid, pl.ds(sid * 8, 8)])


_sc_aggregate = pl.kernel(
    _sc_body,
    out_type=(
        jax.ShapeDtypeStruct((2, N_GRAPHS, D_FEAT), jnp.float32),
        jax.ShapeDtypeStruct((2, EROWS, 128), jnp.float32),
    ),
    mesh=plsc.VectorSubcoreMesh(core_axis_name="c", subcore_axis_name="s"),
    compiler_params=pltpu.CompilerParams(needs_layout_passes=False,
                                         use_tc_tiling_on_sc=True),
    scratch_types=[
        pltpu.VMEM_SHARED((N_GRAPHS, D_FEAT), jnp.float32),
        pltpu.VMEM_SHARED((EROWS, 128), jnp.float32),
        pltpu.VMEM((N_NODES,), jnp.int32),
        pltpu.VMEM((EK,), jnp.int32),
        pltpu.VMEM((D_EDGE, EK), jnp.float32),
        pltpu.VMEM((EK,), jnp.int32),
        pltpu.VMEM((D_EDGE, EK), jnp.float32),
        pltpu.VMEM((EROWS, 128), jnp.float32),
        pltpu.VMEM((NCH,), jnp.int32),
        pltpu.VMEM((NCH, D_FEAT), jnp.float32),
        pltpu.VMEM((NCH,), jnp.int32),
        pltpu.VMEM((NCH, D_FEAT), jnp.float32),
        pltpu.VMEM((N_REM,), jnp.int32),
        pltpu.VMEM((N_REM, D_FEAT), jnp.float32),
        pltpu.VMEM((32, D_FEAT), jnp.float32),
        pltpu.VMEM((EROWS,), jnp.int32),
        pltpu.SemaphoreType.DMA,
        pltpu.SemaphoreType.DMA,
        pltpu.SemaphoreType.DMA,
        pltpu.SemaphoreType.DMA,
        pltpu.SemaphoreType.DMA,
        pltpu.SemaphoreType.DMA,
        pltpu.SemaphoreType.DMA,
    ],
)


def _mlp_body(np_ref, et_ref, w1_ref, b1_ref, g1_ref, bt1_ref,
              w2_ref, b2_ref, g2_ref, bt2_ref, w3_ref, b3_ref, o_ref):
    f32 = jnp.float32
    hi = jax.lax.Precision.HIGHEST
    node = np_ref[0] + np_ref[1]
    edge_t = et_ref[...]
    w1 = w1_ref[...]
    h = (jnp.dot(node, w1[:D_FEAT], preferred_element_type=f32, precision=hi)
         + lax.dot_general(edge_t, w1[D_FEAT:],
                           (((0,), (0,)), ((), ())),
                           preferred_element_type=f32, precision=hi)
         + b1_ref[...])
    h = jnp.where(h >= 0, h, 0.01 * h)
    mean = jnp.mean(h, axis=0, keepdims=True)
    d = h - mean
    var = jnp.mean(d * d, axis=0, keepdims=True)
    h = g1_ref[...] * d / jnp.sqrt(var + 1e-5) + bt1_ref[...]

    h = jnp.dot(h, w2_ref[...], preferred_element_type=f32, precision=hi) + b2_ref[...]
    h = jnp.where(h >= 0, h, 0.01 * h)
    mean = jnp.mean(h, axis=0, keepdims=True)
    d = h - mean
    var = jnp.mean(d * d, axis=0, keepdims=True)
    h = g2_ref[...] * d / jnp.sqrt(var + 1e-5) + bt2_ref[...]

    o_ref[...] = jnp.dot(h, w3_ref[...], preferred_element_type=f32,
                         precision=hi) + b3_ref[...]


_mlp_head = pl.pallas_call(
    _mlp_body,
    out_shape=jax.ShapeDtypeStruct((N_GRAPHS, D_FEAT), jnp.float32),
)


@functools.partial(jax.jit, static_argnames=())
def kernel(x, edge_index, edge_attr, u, batch, W1, b1, g1, beta1,
           W2, b2, g2, beta2, W3, b3):
    col = edge_index[1].astype(jnp.int32)
    batch32 = batch.astype(jnp.int32)
    ea_t = edge_attr.T
    node_part, edge_part = _sc_aggregate(x, batch32, col, ea_t)
    edge_t = (edge_part[0] + edge_part[1]).reshape(D_EDGE, N_GRAPHS)
    return _mlp_head(node_part, edge_t,
                     W1, b1.reshape(1, -1), g1.reshape(1, -1),
                     beta1.reshape(1, -1),
                     W2, b2.reshape(1, -1), g2.reshape(1, -1),
                     beta2.reshape(1, -1),
                     W3, b3.reshape(1, -1))
